# Initial kernel scaffold; baseline (speedup 1.0000x reference)
import jax
import jax.numpy as jnp
from jax.experimental import pallas as pl


def kernel(x, table):
    raise NotImplementedError("write your pallas kernel here")


# speedup vs baseline: 1.1710x; 1.1710x over previous
import functools

import jax
import jax.numpy as jnp
from jax import lax
from jax.experimental import pallas as pl
from jax.experimental.pallas import tpu as pltpu
from jax.experimental.pallas import tpu_sc as plsc

_GRID = (100, 100, 100)
_D = 32
_NUM_EMB = _GRID[0] * _GRID[1] * _GRID[2]
_N = 65536

_NC = 2
_NS = 16
_NW = _NC * _NS
_PW = _N // _NW
_CH = 128
_NCHUNK = _PW // _CH
_G = _CH // 16

_G0 = _GRID[0]
_G01 = _GRID[0] * _GRID[1]
_CORNERS = (
    (0, 0, 0, 0),
    (1, 0, 0, 1),
    (_G0, 0, 1, 0),
    (_G01, 1, 0, 0),
    (_G0 + 1, 0, 1, 1),
    (_G01 + 1, 1, 0, 1),
    (_G01 + _G0, 1, 1, 0),
    (_G01 + _G0 + 1, 1, 1, 1),
)


def _body(xt_hbm, table_hbm, out_hbm, xv, idxv, wv, rows, outv, sem):
    wid = lax.axis_index("c") * _NS + lax.axis_index("s")

    def chunk(g, _):
        base = wid * _PW + g * _CH

        pltpu.sync_copy(xt_hbm.at[:, pl.ds(base, _CH)], xv)

        def grp(i, _):
            s = pl.ds(i * 16, 16)
            rx = xv[0, s] * float(_G0)
            ry = xv[1, s] * float(_G0)
            rz = xv[2, s] * float(_G0)
            ix = rx.astype(jnp.int32)
            iy = ry.astype(jnp.int32)
            iz = rz.astype(jnp.int32)
            fx = rx - ix.astype(jnp.float32)
            fy = ry - iy.astype(jnp.float32)
            fz = rz - iz.astype(jnp.float32)
            c000 = ix + iy * _G0 + iz * _G01
            c000 = jnp.where(c000 >= _NUM_EMB, _NUM_EMB - 1, c000)
            ox = (1.0 - fx, fx)
            oy = (1.0 - fy, fy)
            oz = (1.0 - fz, fz)
            for k, (off, a, b, c) in enumerate(_CORNERS):
                ck = c000 + off
                ck = jnp.where(ck >= _NUM_EMB, c000, ck)
                idxv[k, s] = ck
                wv[k, s] = ox[a] * oy[b] * oz[c]
            return _

        lax.fori_loop(0, _G, grp, None, unroll=False)

        handles = [
            pltpu.async_copy(table_hbm.at[idxv.at[k]], rows.at[k], sem)
            for k in range(8)
        ]
        for h in handles:
            h.wait()

        def cmb(i, _):
            s = pl.ds(i * 16, 16)
            lo = pl.ds(0, 16)
            hi = pl.ds(16, 16)
            wvecs = [wv[k, s] for k in range(8)]
            for j in range(16):
                p = i * 16 + j
                w = wvecs[0][j]
                a0 = rows[0, p, lo] * w
                a1 = rows[0, p, hi] * w
                for k in range(1, 8):
                    w = wvecs[k][j]
                    a0 = a0 + rows[k, p, lo] * w
                    a1 = a1 + rows[k, p, hi] * w
                outv[p, lo] = a0
                outv[p, hi] = a1
            return _

        lax.fori_loop(0, _G, cmb, None, unroll=False)

        pltpu.sync_copy(outv, out_hbm.at[pl.ds(base, _CH)])
        return _

    lax.fori_loop(0, _NCHUNK, chunk, None, unroll=False)


@jax.jit
def _grid_embed(xt, table):
    f = functools.partial(
        pl.kernel,
        mesh=plsc.VectorSubcoreMesh(core_axis_name="c", subcore_axis_name="s"),
        out_type=jax.ShapeDtypeStruct((_N, _D), jnp.float32),
        scratch_types=[
            pltpu.VMEM((3, _CH), jnp.float32),
            pltpu.VMEM((8, _CH), jnp.int32),
            pltpu.VMEM((8, _CH), jnp.float32),
            pltpu.VMEM((8, _CH, _D), jnp.float32),
            pltpu.VMEM((_CH, _D), jnp.float32),
            pltpu.SemaphoreType.DMA,
        ],
        compiler_params=pltpu.CompilerParams(use_tc_tiling_on_sc=False),
    )(_body)
    return f(xt, table)


def kernel(x, table):
    xt = jnp.transpose(x)
    return _grid_embed(xt, table)
